# baseline (speedup 1.0000x reference)
import functools

import jax
import jax.numpy as jnp
from jax import lax
from jax.experimental import pallas as pl
from jax.experimental.pallas import tpu as pltpu
from jax.experimental.pallas import tpu_sc as plsc

NC = 2
NS = 16
K = 128
NI = 3
NB = 2
UNROLL = 6
SHARE0 = 0.77


def _split_rows(n_rows):
    per = -(-n_rows // NS)
    per = -(-per // 8) * 8
    slices = []
    off = 0
    for s in range(NS):
        size = min(per, n_rows - off)
        slices.append((off, max(size, 0)))
        off += size
    return slices


def _sc_agg_call(n, n_pad, d, e_off, spw0, q1, r1):
    mesh = plsc.VectorSubcoreMesh(core_axis_name="c", subcore_axis_name="s")
    rows_per_tile = n_pad // NS
    xslices = _split_rows(n)
    start1 = NS * spw0

    @functools.partial(
        pl.kernel,
        mesh=mesh,
        out_type=jax.ShapeDtypeStruct((NC, n_pad, d), jnp.float32),
        scratch_types=(
            [pltpu.VMEM((K,), jnp.int32) for _ in range(2 * NI)]
            + [pltpu.VMEM((K, d), jnp.float32) for _ in range(NB)]
            + [pltpu.VMEM_SHARED((n_pad, d), jnp.float32)]
            + [pltpu.SemaphoreType.DMA for _ in range(NI + NB + 1)]
        ),
    )
    def sc_agg(x_hbm, se_hbm, de_hbm, out_hbm,
               src0, src1, src2, dst0, dst1, dst2, rows0, rows1,
               agg_sh, isem0, isem1, isem2, gsem0, gsem1, ssem):
        c = lax.axis_index("c")
        s = lax.axis_index("s")
        srcs = (src0, src1, src2)
        dsts = (dst0, dst1, dst2)
        isems = (isem0, isem1, isem2)
        rows = (rows0, rows1)
        gsems = (gsem0, gsem1)
        rslc = pl.ds(s * rows_per_tile, rows_per_tile)
        base0 = s * (spw0 * K)
        cnt1 = q1 + jnp.where(s < r1, 1, 0)
        base1 = (start1 + q1 * s + jnp.minimum(s, r1)) * K

        def idx_start(base, g, j):
            off = pl.multiple_of(base + g * K, K)
            pltpu.async_copy(se_hbm.at[pl.ds(off, K)], srcs[j], isems[j])
            pltpu.async_copy(de_hbm.at[pl.ds(off, K)], dsts[j], isems[j])

        def idx_wait(base, g, j):
            off = pl.multiple_of(base + g * K, K)
            pltpu.make_async_copy(se_hbm.at[pl.ds(off, K)], srcs[j],
                                  isems[j]).wait()
            pltpu.make_async_copy(de_hbm.at[pl.ds(off, K)], dsts[j],
                                  isems[j]).wait()

        def gather_start(j, b):
            pltpu.async_copy(x_hbm.at[srcs[j]], rows[b], gsems[b])

        def gather_wait(j, b):
            pltpu.make_async_copy(x_hbm.at[srcs[j]], rows[b],
                                  gsems[b]).wait()

        @pl.when(c == 0)
        def _():
            for g in range(NI):
                idx_start(base0, g, g)
            for t, (xo, xs_) in enumerate(xslices):
                if xs_ > 0:
                    @pl.when(s == t)
                    def _():
                        pltpu.async_copy(x_hbm.at[pl.ds(xo, xs_)],
                                         agg_sh.at[pl.ds(xo, xs_)], ssem)
                        pltpu.make_async_copy(
                            x_hbm.at[pl.ds(xo, xs_)],
                            agg_sh.at[pl.ds(xo, xs_)], ssem).wait()

        @pl.when(c == 1)
        def _():
            for t, (xo, xs_) in enumerate(xslices):
                if xs_ > 0:
                    @pl.when(s == t)
                    def _():
                        pltpu.async_copy(x_hbm.at[pl.ds(xo, xs_)],
                                         agg_sh.at[pl.ds(xo, xs_)], ssem)
                        pltpu.make_async_copy(
                            x_hbm.at[pl.ds(xo, xs_)],
                            agg_sh.at[pl.ds(xo, xs_)], ssem).wait()

        plsc.subcore_barrier()

        @pl.when(c == 0)
        def _():
            idx_wait(base0, 0, 0)
            gather_start(0, 0)
            idx_wait(base0, 1, 1)
            gather_start(1, 1)

            def step(i, carry):
                g0 = i * UNROLL
                for u in range(UNROLL):
                    g = g0 + u
                    b = u % NB
                    j = u % NI
                    jn = (u + 2) % NI
                    gather_wait(j, b)
                    pltpu.sync_copy(rows[b], agg_sh.at[dsts[j]], add=True)

                    @pl.when(g + NI < spw0)
                    def _():
                        idx_start(base0, g + NI, j)

                    @pl.when(g + 2 < spw0)
                    def _():
                        idx_wait(base0, g + 2, jn)
                        gather_start(jn, b)
                return carry

            lax.fori_loop(0, spw0 // UNROLL, step, 0)

        @pl.when(c == 1)
        def _():
            def step(g, carry):
                off = pl.multiple_of(base1 + g * K, K)
                pltpu.sync_copy(se_hbm.at[pl.ds(off, K)], srcs[0])
                pltpu.sync_copy(de_hbm.at[pl.ds(off, K)], dsts[0])
                gather_start(0, 0)
                gather_wait(0, 0)
                pltpu.sync_copy(rows[0], agg_sh.at[dsts[0]], add=True)
                return carry

            lax.fori_loop(0, cnt1, step, 0)

        plsc.subcore_barrier()
        pltpu.sync_copy(agg_sh.at[rslc], out_hbm.at[c, rslc])

    return sc_agg


def _dense_body(n, sr, xr, w1r, b1r, w2r, b2r, gr, br, outr):
    h = sr[0, :n, :] + sr[1, :n, :] - xr[...]
    a = jnp.dot(h, w1r[...], preferred_element_type=jnp.float32) + b1r[...]
    a = jnp.maximum(a, 0.0)
    h2 = jnp.dot(a, w2r[...], preferred_element_type=jnp.float32) + b2r[...]
    mean = jnp.mean(h2, axis=0, keepdims=True)
    cent = h2 - mean
    var = jnp.mean(cent * cent, axis=0, keepdims=True)
    scale = lax.rsqrt(var + 1e-5) * gr[...]
    outr[...] = jnp.maximum(cent * scale + br[...], 0.0)


def kernel(x, edge_index, W1, b1, W2, b2, gamma, beta):
    n, d = x.shape
    e = edge_index.shape[1]
    n_pad = -(-n // (NS * 8)) * (NS * 8)

    steps = e // K
    assert e % K == 0 and e % 8 == 0, "edge count must be step-aligned"
    spw0 = int(steps * SHARE0) // NS
    spw0 = (spw0 // UNROLL) * UNROLL
    q1, r1 = divmod(steps - NS * spw0, NS)

    ei = edge_index.astype(jnp.int32)
    se, de = ei[0], ei[1]

    slabs = _sc_agg_call(n, n_pad, d, e, spw0, q1, r1)(x, se, de)

    out = pl.pallas_call(
        functools.partial(_dense_body, n),
        out_shape=jax.ShapeDtypeStruct((n, d), jnp.float32),
    )(slabs, x, W1.T, b1.reshape(1, d), W2.T,
      b2.reshape(1, d), gamma.reshape(1, d), beta.reshape(1, d))
    return out


# speedup vs baseline: 1.0408x; 1.0408x over previous
import functools

import jax
import jax.numpy as jnp
from jax import lax
from jax.experimental import pallas as pl
from jax.experimental.pallas import tpu as pltpu
from jax.experimental.pallas import tpu_sc as plsc

NC = 2
NS = 16
K = 128
NI = 3
NB = 2
UNROLL = 6
B1 = 8
SHARE0 = 0.77


def _split_rows(n_rows):
    per = -(-n_rows // NS)
    per = -(-per // 8) * 8
    slices = []
    off = 0
    for s in range(NS):
        size = min(per, n_rows - off)
        slices.append((off, max(size, 0)))
        off += size
    return slices


def _sc_agg_call(n, n_pad, d, e_off, spw0, q1, r1):
    mesh = plsc.VectorSubcoreMesh(core_axis_name="c", subcore_axis_name="s")
    rows_per_tile = n_pad // NS
    xslices = _split_rows(n)
    start1 = NS * spw0

    @functools.partial(
        pl.kernel,
        mesh=mesh,
        out_type=jax.ShapeDtypeStruct((NC, n_pad, d), jnp.float32),
        scratch_types=(
            [pltpu.VMEM((K,), jnp.int32) for _ in range(2 * NI)]
            + [pltpu.VMEM((B1 * K,), jnp.int32) for _ in range(2)]
            + [pltpu.VMEM((K, d), jnp.float32) for _ in range(NB)]
            + [pltpu.VMEM_SHARED((n_pad, d), jnp.float32)]
            + [pltpu.SemaphoreType.DMA for _ in range(NI + NB + 1)]
        ),
    )
    def sc_agg(x_hbm, zz_hbm, ei_hbm, out_hbm,
               src0, src1, src2, dst0, dst1, dst2, sblk, dblk, rows0, rows1,
               agg_sh, isem0, isem1, isem2, gsem0, gsem1, ssem):
        c = lax.axis_index("c")
        s = lax.axis_index("s")
        srcs = (src0, src1, src2)
        dsts = (dst0, dst1, dst2)
        isems = (isem0, isem1, isem2)
        rows = (rows0, rows1)
        gsems = (gsem0, gsem1)
        rslc = pl.ds(s * rows_per_tile, rows_per_tile)
        base0 = s * (spw0 * K)
        cnt1 = q1 + jnp.where(s < r1, 1, 0)
        base1 = (start1 + q1 * s + jnp.minimum(s, r1)) * K

        def idx_start(base, g, j):
            off = pl.multiple_of(base + g * K, K)
            pltpu.async_copy(ei_hbm.at[pl.ds(off, K)], srcs[j], isems[j])
            pltpu.async_copy(ei_hbm.at[pl.ds(e_off + off, K)], dsts[j],
                             isems[j])

        def idx_wait(base, g, j):
            off = pl.multiple_of(base + g * K, K)
            pltpu.make_async_copy(ei_hbm.at[pl.ds(off, K)], srcs[j],
                                  isems[j]).wait()
            pltpu.make_async_copy(ei_hbm.at[pl.ds(e_off + off, K)], dsts[j],
                                  isems[j]).wait()

        def gather_start(j, b):
            pltpu.async_copy(x_hbm.at[srcs[j]], rows[b], gsems[b])

        def gather_wait(j, b):
            pltpu.make_async_copy(x_hbm.at[srcs[j]], rows[b],
                                  gsems[b]).wait()

        @pl.when(c == 0)
        def _():
            for g in range(NI):
                idx_start(base0, g, g)
            for t, (xo, xs_) in enumerate(xslices):
                if xs_ > 0:
                    @pl.when(s == t)
                    def _():
                        pltpu.async_copy(x_hbm.at[pl.ds(xo, xs_)],
                                         agg_sh.at[pl.ds(xo, xs_)], ssem)
                        pltpu.make_async_copy(
                            x_hbm.at[pl.ds(xo, xs_)],
                            agg_sh.at[pl.ds(xo, xs_)], ssem).wait()

        @pl.when(c == 1)
        def _():
            pltpu.async_copy(zz_hbm.at[rslc], agg_sh.at[rslc], ssem)
            pltpu.make_async_copy(zz_hbm.at[rslc], agg_sh.at[rslc],
                                  ssem).wait()

        plsc.subcore_barrier()

        @pl.when(c == 0)
        def _():
            idx_wait(base0, 0, 0)
            gather_start(0, 0)
            idx_wait(base0, 1, 1)
            gather_start(1, 1)

            def step(i, carry):
                g0 = i * UNROLL
                for u in range(UNROLL):
                    g = g0 + u
                    b = u % NB
                    j = u % NI
                    jn = (u + 2) % NI
                    gather_wait(j, b)
                    pltpu.sync_copy(rows[b], agg_sh.at[dsts[j]], add=True)

                    @pl.when(g + NI < spw0)
                    def _():
                        idx_start(base0, g + NI, j)

                    @pl.when(g + 2 < spw0)
                    def _():
                        idx_wait(base0, g + 2, jn)
                        gather_start(jn, b)
                return carry

            lax.fori_loop(0, spw0 // UNROLL, step, 0)

        @pl.when(c == 1)
        def _():
            def blk(i, carry):
                boff = pl.multiple_of(base1 + i * (B1 * K), K)
                pltpu.sync_copy(ei_hbm.at[pl.ds(boff, B1 * K)], sblk)
                pltpu.sync_copy(ei_hbm.at[pl.ds(e_off + boff, B1 * K)], dblk)
                for u in range(B1):
                    uslc = pl.ds(u * K, K)
                    for v in range(K // 16):
                        dsts[0][pl.ds(v * 16, 16)] = (
                            dblk[pl.ds(u * K + v * 16, 16)])
                    pltpu.async_copy(x_hbm.at[sblk.at[uslc]], rows[0],
                                     gsems[0])
                    pltpu.make_async_copy(x_hbm.at[sblk.at[uslc]], rows[0],
                                          gsems[0]).wait()
                    pltpu.sync_copy(rows[0], agg_sh.at[dsts[0]], add=True)
                return carry

            nblk = cnt1 // B1
            lax.fori_loop(0, nblk, blk, 0)

            def step(g, carry):
                off = pl.multiple_of(base1 + g * K, K)
                pltpu.sync_copy(ei_hbm.at[pl.ds(off, K)], srcs[0])
                pltpu.sync_copy(ei_hbm.at[pl.ds(e_off + off, K)], dsts[0])
                gather_start(0, 0)
                gather_wait(0, 0)
                pltpu.sync_copy(rows[0], agg_sh.at[dsts[0]], add=True)
                return carry

            lax.fori_loop(nblk * B1, cnt1, step, 0)

        plsc.subcore_barrier()
        pltpu.sync_copy(agg_sh.at[rslc], out_hbm.at[c, rslc])

    return sc_agg


def _dense_body(n, sr, w1r, b1r, w2r, b2r, gr, br, outr):
    h = sr[0, :n, :] + sr[1, :n, :]
    a = jnp.dot(h, w1r[...], preferred_element_type=jnp.float32) + b1r[...]
    a = jnp.maximum(a, 0.0)
    h2 = jnp.dot(a, w2r[...], preferred_element_type=jnp.float32) + b2r[...]
    mean = jnp.mean(h2, axis=0, keepdims=True)
    cent = h2 - mean
    var = jnp.mean(cent * cent, axis=0, keepdims=True)
    scale = lax.rsqrt(var + 1e-5) * gr[...]
    outr[...] = jnp.maximum(cent * scale + br[...], 0.0)


def kernel(x, edge_index, W1, b1, W2, b2, gamma, beta):
    n, d = x.shape
    e = edge_index.shape[1]
    n_pad = -(-n // (NS * 8)) * (NS * 8)

    steps = e // K
    assert e % K == 0 and e % 8 == 0, "edge count must be step-aligned"
    spw0 = int(steps * SHARE0) // NS
    spw0 = (spw0 // UNROLL) * UNROLL
    q1, r1 = divmod(steps - NS * spw0, NS)

    ei = edge_index.astype(jnp.int32).reshape(2 * e)
    zz = jnp.zeros((n_pad, d), jnp.float32)

    slabs = _sc_agg_call(n, n_pad, d, e, spw0, q1, r1)(x, zz, ei)

    out = pl.pallas_call(
        functools.partial(_dense_body, n),
        out_shape=jax.ShapeDtypeStruct((n, d), jnp.float32),
    )(slabs, W1.T, b1.reshape(1, d), W2.T,
      b2.reshape(1, d), gamma.reshape(1, d), beta.reshape(1, d))
    return out


# speedup vs baseline: 1.1359x; 1.0914x over previous
import functools

import jax
import jax.numpy as jnp
from jax import lax
from jax.experimental import pallas as pl
from jax.experimental.pallas import tpu as pltpu
from jax.experimental.pallas import tpu_sc as plsc

NC = 2
NS = 16
K = 128
NI = 3
NB = 2
UNROLL = 6
B1 = 8
SHARE0 = 0.692


def _split_rows(n_rows):
    per = -(-n_rows // NS)
    per = -(-per // 8) * 8
    slices = []
    off = 0
    for s in range(NS):
        size = min(per, n_rows - off)
        slices.append((off, max(size, 0)))
        off += size
    return slices


def _sc_agg_call(n, n_pad, d, e_off, spw0, q1, r1):
    mesh = plsc.VectorSubcoreMesh(core_axis_name="c", subcore_axis_name="s")
    rows_per_tile = n_pad // NS
    xslices = _split_rows(n)
    start1 = NS * spw0

    @functools.partial(
        pl.kernel,
        mesh=mesh,
        out_type=jax.ShapeDtypeStruct((NC, n_pad, d), jnp.float32),
        scratch_types=(
            [pltpu.VMEM((K,), jnp.int32) for _ in range(2 * NI)]
            + [pltpu.VMEM((B1 * K,), jnp.int32) for _ in range(2)]
            + [pltpu.VMEM((K, d), jnp.float32) for _ in range(NB)]
            + [pltpu.VMEM_SHARED((n_pad, d), jnp.float32)]
            + [pltpu.SemaphoreType.DMA for _ in range(NI + NB + 1)]
        ),
    )
    def sc_agg(x_hbm, zz_hbm, ei_hbm, out_hbm,
               src0, src1, src2, dst0, dst1, dst2, sblk, dblk, rows0, rows1,
               agg_sh, isem0, isem1, isem2, gsem0, gsem1, ssem):
        c = lax.axis_index("c")
        s = lax.axis_index("s")
        srcs = (src0, src1, src2)
        dsts = (dst0, dst1, dst2)
        isems = (isem0, isem1, isem2)
        rows = (rows0, rows1)
        gsems = (gsem0, gsem1)
        rslc = pl.ds(s * rows_per_tile, rows_per_tile)
        base0 = s * (spw0 * K)
        cnt1 = q1 + jnp.where(s < r1, 1, 0)
        base1 = (start1 + q1 * s + jnp.minimum(s, r1)) * K

        def idx_start(base, g, j):
            off = pl.multiple_of(base + g * K, K)
            pltpu.async_copy(ei_hbm.at[pl.ds(off, K)], srcs[j], isems[j])
            pltpu.async_copy(ei_hbm.at[pl.ds(e_off + off, K)], dsts[j],
                             isems[j])

        def idx_wait(base, g, j):
            off = pl.multiple_of(base + g * K, K)
            pltpu.make_async_copy(ei_hbm.at[pl.ds(off, K)], srcs[j],
                                  isems[j]).wait()
            pltpu.make_async_copy(ei_hbm.at[pl.ds(e_off + off, K)], dsts[j],
                                  isems[j]).wait()

        def gather_start(j, b):
            pltpu.async_copy(x_hbm.at[srcs[j]], rows[b], gsems[b])

        def gather_wait(j, b):
            pltpu.make_async_copy(x_hbm.at[srcs[j]], rows[b],
                                  gsems[b]).wait()

        @pl.when(c == 0)
        def _():
            for g in range(NI):
                idx_start(base0, g, g)
            for t, (xo, xs_) in enumerate(xslices):
                if xs_ > 0:
                    @pl.when(s == t)
                    def _():
                        pltpu.async_copy(x_hbm.at[pl.ds(xo, xs_)],
                                         agg_sh.at[pl.ds(xo, xs_)], ssem)
                        pltpu.make_async_copy(
                            x_hbm.at[pl.ds(xo, xs_)],
                            agg_sh.at[pl.ds(xo, xs_)], ssem).wait()

        @pl.when(c == 1)
        def _():
            pltpu.async_copy(zz_hbm.at[rslc], agg_sh.at[rslc], ssem)
            pltpu.make_async_copy(zz_hbm.at[rslc], agg_sh.at[rslc],
                                  ssem).wait()

        plsc.subcore_barrier()

        @pl.when(c == 0)
        def _():
            idx_wait(base0, 0, 0)
            gather_start(0, 0)
            idx_wait(base0, 1, 1)
            gather_start(1, 1)

            def step(i, carry):
                g0 = i * UNROLL
                for u in range(UNROLL):
                    g = g0 + u
                    b = u % NB
                    j = u % NI
                    jn = (u + 2) % NI
                    gather_wait(j, b)
                    pltpu.sync_copy(rows[b], agg_sh.at[dsts[j]], add=True)

                    @pl.when(g + NI < spw0)
                    def _():
                        idx_start(base0, g + NI, j)

                    @pl.when(g + 2 < spw0)
                    def _():
                        idx_wait(base0, g + 2, jn)
                        gather_start(jn, b)
                return carry

            lax.fori_loop(0, spw0 // UNROLL, step, 0)

        @pl.when(c == 1)
        def _():
            def blk(i, carry):
                boff = pl.multiple_of(base1 + i * (B1 * K), K)
                pltpu.sync_copy(ei_hbm.at[pl.ds(boff, B1 * K)], sblk)
                pltpu.sync_copy(ei_hbm.at[pl.ds(e_off + boff, B1 * K)], dblk)
                for u in range(B1):
                    uslc = pl.ds(u * K, K)
                    for v in range(K // 16):
                        dsts[0][pl.ds(v * 16, 16)] = (
                            dblk[pl.ds(u * K + v * 16, 16)])
                    pltpu.async_copy(x_hbm.at[sblk.at[uslc]], rows[0],
                                     gsems[0])
                    pltpu.make_async_copy(x_hbm.at[sblk.at[uslc]], rows[0],
                                          gsems[0]).wait()
                    pltpu.sync_copy(rows[0], agg_sh.at[dsts[0]], add=True)
                return carry

            nblk = cnt1 // B1
            lax.fori_loop(0, nblk, blk, 0)

            def step(g, carry):
                off = pl.multiple_of(base1 + g * K, K)
                pltpu.sync_copy(ei_hbm.at[pl.ds(off, K)], srcs[0])
                pltpu.sync_copy(ei_hbm.at[pl.ds(e_off + off, K)], dsts[0])
                gather_start(0, 0)
                gather_wait(0, 0)
                pltpu.sync_copy(rows[0], agg_sh.at[dsts[0]], add=True)
                return carry

            lax.fori_loop(nblk * B1, cnt1, step, 0)

        plsc.subcore_barrier()
        pltpu.sync_copy(agg_sh.at[rslc], out_hbm.at[c, rslc])

    return sc_agg


def _dense_body(n, sr, w1r, b1r, w2r, b2r, gr, br, outr):
    h = sr[0, :n, :] + sr[1, :n, :]
    a = jnp.dot(h, w1r[...], preferred_element_type=jnp.float32) + b1r[...]
    a = jnp.maximum(a, 0.0)
    h2 = jnp.dot(a, w2r[...], preferred_element_type=jnp.float32) + b2r[...]
    mean = jnp.mean(h2, axis=0, keepdims=True)
    cent = h2 - mean
    var = jnp.mean(cent * cent, axis=0, keepdims=True)
    scale = lax.rsqrt(var + 1e-5) * gr[...]
    outr[...] = jnp.maximum(cent * scale + br[...], 0.0)


def kernel(x, edge_index, W1, b1, W2, b2, gamma, beta):
    n, d = x.shape
    e = edge_index.shape[1]
    n_pad = -(-n // (NS * 8)) * (NS * 8)

    steps = e // K
    assert e % K == 0 and e % 8 == 0, "edge count must be step-aligned"
    spw0 = int(steps * SHARE0) // NS
    spw0 = (spw0 // UNROLL) * UNROLL
    q1, r1 = divmod(steps - NS * spw0, NS)

    ei = edge_index.astype(jnp.int32).reshape(2 * e)
    zz = jnp.zeros((n_pad, d), jnp.float32)

    slabs = _sc_agg_call(n, n_pad, d, e, spw0, q1, r1)(x, zz, ei)

    out = pl.pallas_call(
        functools.partial(_dense_body, n),
        out_shape=jax.ShapeDtypeStruct((n, d), jnp.float32),
    )(slabs, W1.T, b1.reshape(1, d), W2.T,
      b2.reshape(1, d), gamma.reshape(1, d), beta.reshape(1, d))
    return out


# speedup vs baseline: 1.1581x; 1.0195x over previous
import functools

import jax
import jax.numpy as jnp
from jax import lax
from jax.experimental import pallas as pl
from jax.experimental.pallas import tpu as pltpu
from jax.experimental.pallas import tpu_sc as plsc

NC = 2
NS = 16
K = 128
NI = 3
NB = 2
UNROLL = 6
B1 = 8
SHARE0 = 0.692


def _split_rows(n_rows):
    per = -(-n_rows // NS)
    per = -(-per // 8) * 8
    slices = []
    off = 0
    for s in range(NS):
        size = min(per, n_rows - off)
        slices.append((off, max(size, 0)))
        off += size
    return slices


def _sc_agg_call(n, n_pad, d, e_off, spw0, q1, r1):
    mesh = plsc.VectorSubcoreMesh(core_axis_name="c", subcore_axis_name="s")
    rows_per_tile = n_pad // NS
    xslices = _split_rows(n)
    start1 = NS * spw0

    @functools.partial(
        pl.kernel,
        mesh=mesh,
        out_type=jax.ShapeDtypeStruct((NC, n_pad, d), jnp.float32),
        scratch_types=(
            [pltpu.VMEM((K,), jnp.int32) for _ in range(2 * NI)]
            + [pltpu.VMEM((B1 * K,), jnp.int32) for _ in range(2)]
            + [pltpu.VMEM((K, d), jnp.float32) for _ in range(NB)]
            + [pltpu.VMEM_SHARED((n_pad, d), jnp.float32)]
            + [pltpu.SemaphoreType.DMA for _ in range(NI + NB + 1)]
        ),
    )
    def sc_agg(x_hbm, zz_hbm, ei_hbm, out_hbm,
               src0, src1, src2, dst0, dst1, dst2, sblk, dblk, rows0, rows1,
               agg_sh, isem0, isem1, isem2, gsem0, gsem1, ssem):
        c = lax.axis_index("c")
        s = lax.axis_index("s")
        srcs = (src0, src1, src2)
        dsts = (dst0, dst1, dst2)
        isems = (isem0, isem1, isem2)
        rows = (rows0, rows1)
        gsems = (gsem0, gsem1)
        rslc = pl.ds(s * rows_per_tile, rows_per_tile)
        base0 = s * (spw0 * K)
        cnt1 = q1 + jnp.where(s < r1, 1, 0)
        base1 = (start1 + q1 * s + jnp.minimum(s, r1)) * K

        def idx_start(base, g, j):
            off = pl.multiple_of(base + g * K, K)
            pltpu.async_copy(ei_hbm.at[0, pl.ds(off, K)], srcs[j], isems[j])
            pltpu.async_copy(ei_hbm.at[1, pl.ds(off, K)], dsts[j],
                             isems[j])

        def idx_wait(base, g, j):
            off = pl.multiple_of(base + g * K, K)
            pltpu.make_async_copy(ei_hbm.at[0, pl.ds(off, K)], srcs[j],
                                  isems[j]).wait()
            pltpu.make_async_copy(ei_hbm.at[1, pl.ds(off, K)], dsts[j],
                                  isems[j]).wait()

        def gather_start(j, b):
            pltpu.async_copy(x_hbm.at[srcs[j]], rows[b], gsems[b])

        def gather_wait(j, b):
            pltpu.make_async_copy(x_hbm.at[srcs[j]], rows[b],
                                  gsems[b]).wait()

        @pl.when(c == 0)
        def _():
            for g in range(NI):
                idx_start(base0, g, g)
            for t, (xo, xs_) in enumerate(xslices):
                if xs_ > 0:
                    @pl.when(s == t)
                    def _():
                        pltpu.async_copy(x_hbm.at[pl.ds(xo, xs_)],
                                         agg_sh.at[pl.ds(xo, xs_)], ssem)
                        pltpu.make_async_copy(
                            x_hbm.at[pl.ds(xo, xs_)],
                            agg_sh.at[pl.ds(xo, xs_)], ssem).wait()

        @pl.when(c == 1)
        def _():
            pltpu.async_copy(zz_hbm.at[rslc], agg_sh.at[rslc], ssem)
            pltpu.make_async_copy(zz_hbm.at[rslc], agg_sh.at[rslc],
                                  ssem).wait()

        plsc.subcore_barrier()

        @pl.when(c == 0)
        def _():
            idx_wait(base0, 0, 0)
            gather_start(0, 0)
            idx_wait(base0, 1, 1)
            gather_start(1, 1)

            def step(i, carry):
                g0 = i * UNROLL
                for u in range(UNROLL):
                    g = g0 + u
                    b = u % NB
                    j = u % NI
                    jn = (u + 2) % NI
                    gather_wait(j, b)
                    pltpu.sync_copy(rows[b], agg_sh.at[dsts[j]], add=True)

                    @pl.when(g + NI < spw0)
                    def _():
                        idx_start(base0, g + NI, j)

                    @pl.when(g + 2 < spw0)
                    def _():
                        idx_wait(base0, g + 2, jn)
                        gather_start(jn, b)
                return carry

            lax.fori_loop(0, spw0 // UNROLL, step, 0)

        @pl.when(c == 1)
        def _():
            def blk(i, carry):
                boff = pl.multiple_of(base1 + i * (B1 * K), K)
                pltpu.sync_copy(ei_hbm.at[0, pl.ds(boff, B1 * K)], sblk)
                pltpu.sync_copy(ei_hbm.at[1, pl.ds(boff, B1 * K)], dblk)
                for u in range(B1):
                    uslc = pl.ds(u * K, K)
                    for v in range(K // 16):
                        dsts[0][pl.ds(v * 16, 16)] = (
                            dblk[pl.ds(u * K + v * 16, 16)])
                    pltpu.async_copy(x_hbm.at[sblk.at[uslc]], rows[0],
                                     gsems[0])
                    pltpu.make_async_copy(x_hbm.at[sblk.at[uslc]], rows[0],
                                          gsems[0]).wait()
                    pltpu.sync_copy(rows[0], agg_sh.at[dsts[0]], add=True)
                return carry

            nblk = cnt1 // B1
            lax.fori_loop(0, nblk, blk, 0)

            def step(g, carry):
                off = pl.multiple_of(base1 + g * K, K)
                pltpu.sync_copy(ei_hbm.at[0, pl.ds(off, K)], srcs[0])
                pltpu.sync_copy(ei_hbm.at[1, pl.ds(off, K)], dsts[0])
                gather_start(0, 0)
                gather_wait(0, 0)
                pltpu.sync_copy(rows[0], agg_sh.at[dsts[0]], add=True)
                return carry

            lax.fori_loop(nblk * B1, cnt1, step, 0)

        plsc.subcore_barrier()
        pltpu.sync_copy(agg_sh.at[rslc], out_hbm.at[c, rslc])

    return sc_agg


def _dense_body(n, sr, w1r, b1r, w2r, b2r, gr, br, outr):
    h = sr[0, :n, :] + sr[1, :n, :]
    a = jnp.dot(h, w1r[...], preferred_element_type=jnp.float32) + b1r[...]
    a = jnp.maximum(a, 0.0)
    h2 = jnp.dot(a, w2r[...], preferred_element_type=jnp.float32) + b2r[...]
    mean = jnp.mean(h2, axis=0, keepdims=True)
    cent = h2 - mean
    var = jnp.mean(cent * cent, axis=0, keepdims=True)
    scale = lax.rsqrt(var + 1e-5) * gr[...]
    outr[...] = jnp.maximum(cent * scale + br[...], 0.0)


def kernel(x, edge_index, W1, b1, W2, b2, gamma, beta):
    n, d = x.shape
    e = edge_index.shape[1]
    n_pad = -(-n // (NS * 8)) * (NS * 8)

    steps = e // K
    assert e % K == 0 and e % 8 == 0, "edge count must be step-aligned"
    spw0 = int(steps * SHARE0) // NS
    spw0 = (spw0 // UNROLL) * UNROLL
    q1, r1 = divmod(steps - NS * spw0, NS)

    ei = edge_index.astype(jnp.int32)
    zz = jnp.zeros((n_pad, d), jnp.float32)

    slabs = _sc_agg_call(n, n_pad, d, e, spw0, q1, r1)(x, zz, ei)

    out = pl.pallas_call(
        functools.partial(_dense_body, n),
        out_shape=jax.ShapeDtypeStruct((n, d), jnp.float32),
    )(slabs, W1.T, b1.reshape(1, d), W2.T,
      b2.reshape(1, d), gamma.reshape(1, d), beta.reshape(1, d))
    return out


# speedup vs baseline: 1.1606x; 1.0022x over previous
import functools

import jax
import jax.numpy as jnp
from jax import lax
from jax.experimental import pallas as pl
from jax.experimental.pallas import tpu as pltpu
from jax.experimental.pallas import tpu_sc as plsc

NC = 2
NS = 16
K = 128
NI = 3
NB = 2
UNROLL = 6
B1 = 8
SHARE0 = 0.692


def _split_rows(n_rows):
    per = -(-n_rows // NS)
    per = -(-per // 8) * 8
    slices = []
    off = 0
    for s in range(NS):
        size = min(per, n_rows - off)
        slices.append((off, max(size, 0)))
        off += size
    return slices


def _sc_agg_call(n, n_pad, d, e_off, spw0, q1, r1):
    mesh = plsc.VectorSubcoreMesh(core_axis_name="c", subcore_axis_name="s")
    rows_per_tile = n_pad // NS
    xslices = _split_rows(n)
    start1 = NS * spw0

    @functools.partial(
        pl.kernel,
        mesh=mesh,
        out_type=jax.ShapeDtypeStruct((NC, n_pad, d), jnp.float32),
        scratch_types=(
            [pltpu.VMEM((K,), jnp.int32) for _ in range(2 * NI)]
            + [pltpu.VMEM((B1 * K,), jnp.int32) for _ in range(2)]
            + [pltpu.VMEM((K, d), jnp.float32) for _ in range(NB)]
            + [pltpu.VMEM_SHARED((n_pad, d), jnp.float32)]
            + [pltpu.SemaphoreType.DMA for _ in range(NI + NB + 1)]
        ),
    )
    def sc_agg(x_hbm, ei_hbm, out_hbm,
               src0, src1, src2, dst0, dst1, dst2, sblk, dblk, rows0, rows1,
               agg_sh, isem0, isem1, isem2, gsem0, gsem1, ssem):
        c = lax.axis_index("c")
        s = lax.axis_index("s")
        srcs = (src0, src1, src2)
        dsts = (dst0, dst1, dst2)
        isems = (isem0, isem1, isem2)
        rows = (rows0, rows1)
        gsems = (gsem0, gsem1)
        rslc = pl.ds(s * rows_per_tile, rows_per_tile)
        base0 = s * (spw0 * K)
        cnt1 = q1 + jnp.where(s < r1, 1, 0)
        base1 = (start1 + q1 * s + jnp.minimum(s, r1)) * K

        def idx_start(base, g, j):
            off = pl.multiple_of(base + g * K, K)
            pltpu.async_copy(ei_hbm.at[0, pl.ds(off, K)], srcs[j], isems[j])
            pltpu.async_copy(ei_hbm.at[1, pl.ds(off, K)], dsts[j],
                             isems[j])

        def idx_wait(base, g, j):
            off = pl.multiple_of(base + g * K, K)
            pltpu.make_async_copy(ei_hbm.at[0, pl.ds(off, K)], srcs[j],
                                  isems[j]).wait()
            pltpu.make_async_copy(ei_hbm.at[1, pl.ds(off, K)], dsts[j],
                                  isems[j]).wait()

        def gather_start(j, b):
            pltpu.async_copy(x_hbm.at[srcs[j]], rows[b], gsems[b])

        def gather_wait(j, b):
            pltpu.make_async_copy(x_hbm.at[srcs[j]], rows[b],
                                  gsems[b]).wait()

        @pl.when(c == 0)
        def _():
            for g in range(NI):
                idx_start(base0, g, g)
            for t, (xo, xs_) in enumerate(xslices):
                if xs_ > 0:
                    @pl.when(s == t)
                    def _():
                        pltpu.async_copy(x_hbm.at[pl.ds(xo, xs_)],
                                         agg_sh.at[pl.ds(xo, xs_)], ssem)
                        pltpu.make_async_copy(
                            x_hbm.at[pl.ds(xo, xs_)],
                            agg_sh.at[pl.ds(xo, xs_)], ssem).wait()

        @pl.when(c == 1)
        def _():
            def zrow(r, carry):
                for v in range(K // 16):
                    rows1[r, pl.ds(v * 16, 16)] = jnp.zeros(
                        (16,), jnp.float32)
                return carry

            lax.fori_loop(0, K, zrow, 0)
            full = rows_per_tile // K
            remr = rows_per_tile - full * K
            for fb in range(full):
                pltpu.sync_copy(
                    rows1,
                    agg_sh.at[pl.ds(s * rows_per_tile + fb * K, K)])
            if remr:
                pltpu.sync_copy(
                    rows1.at[pl.ds(0, remr)],
                    agg_sh.at[pl.ds(s * rows_per_tile + full * K, remr)])

        plsc.subcore_barrier()

        @pl.when(c == 0)
        def _():
            idx_wait(base0, 0, 0)
            gather_start(0, 0)
            idx_wait(base0, 1, 1)
            gather_start(1, 1)

            def step(i, carry):
                g0 = i * UNROLL
                for u in range(UNROLL):
                    g = g0 + u
                    b = u % NB
                    j = u % NI
                    jn = (u + 2) % NI
                    gather_wait(j, b)
                    pltpu.sync_copy(rows[b], agg_sh.at[dsts[j]], add=True)

                    @pl.when(g + NI < spw0)
                    def _():
                        idx_start(base0, g + NI, j)

                    @pl.when(g + 2 < spw0)
                    def _():
                        idx_wait(base0, g + 2, jn)
                        gather_start(jn, b)
                return carry

            lax.fori_loop(0, spw0 // UNROLL, step, 0)

        @pl.when(c == 1)
        def _():
            def blk(i, carry):
                boff = pl.multiple_of(base1 + i * (B1 * K), K)
                pltpu.sync_copy(ei_hbm.at[0, pl.ds(boff, B1 * K)], sblk)
                pltpu.sync_copy(ei_hbm.at[1, pl.ds(boff, B1 * K)], dblk)
                for u in range(B1):
                    uslc = pl.ds(u * K, K)
                    for v in range(K // 16):
                        dsts[0][pl.ds(v * 16, 16)] = (
                            dblk[pl.ds(u * K + v * 16, 16)])
                    pltpu.async_copy(x_hbm.at[sblk.at[uslc]], rows[0],
                                     gsems[0])
                    pltpu.make_async_copy(x_hbm.at[sblk.at[uslc]], rows[0],
                                          gsems[0]).wait()
                    pltpu.sync_copy(rows[0], agg_sh.at[dsts[0]], add=True)
                return carry

            nblk = cnt1 // B1
            lax.fori_loop(0, nblk, blk, 0)

            def step(g, carry):
                off = pl.multiple_of(base1 + g * K, K)
                pltpu.sync_copy(ei_hbm.at[0, pl.ds(off, K)], srcs[0])
                pltpu.sync_copy(ei_hbm.at[1, pl.ds(off, K)], dsts[0])
                gather_start(0, 0)
                gather_wait(0, 0)
                pltpu.sync_copy(rows[0], agg_sh.at[dsts[0]], add=True)
                return carry

            lax.fori_loop(nblk * B1, cnt1, step, 0)

        plsc.subcore_barrier()
        pltpu.sync_copy(agg_sh.at[rslc], out_hbm.at[c, rslc])

    return sc_agg


def _dense_body(n, sr, w1r, b1r, w2r, b2r, gr, br, outr):
    h = sr[0, :n, :] + sr[1, :n, :]
    a = jnp.dot(h, w1r[...], preferred_element_type=jnp.float32) + b1r[...]
    a = jnp.maximum(a, 0.0)
    h2 = jnp.dot(a, w2r[...], preferred_element_type=jnp.float32) + b2r[...]
    mean = jnp.mean(h2, axis=0, keepdims=True)
    cent = h2 - mean
    var = jnp.mean(cent * cent, axis=0, keepdims=True)
    scale = lax.rsqrt(var + 1e-5) * gr[...]
    outr[...] = jnp.maximum(cent * scale + br[...], 0.0)


def kernel(x, edge_index, W1, b1, W2, b2, gamma, beta):
    n, d = x.shape
    e = edge_index.shape[1]
    n_pad = -(-n // (NS * 8)) * (NS * 8)

    steps = e // K
    assert e % K == 0 and e % 8 == 0, "edge count must be step-aligned"
    spw0 = int(steps * SHARE0) // NS
    spw0 = (spw0 // UNROLL) * UNROLL
    q1, r1 = divmod(steps - NS * spw0, NS)

    ei = edge_index.astype(jnp.int32)

    slabs = _sc_agg_call(n, n_pad, d, e, spw0, q1, r1)(x, ei)

    out = pl.pallas_call(
        functools.partial(_dense_body, n),
        out_shape=jax.ShapeDtypeStruct((n, d), jnp.float32),
    )(slabs, W1.T, b1.reshape(1, d), W2.T,
      b2.reshape(1, d), gamma.reshape(1, d), beta.reshape(1, d))
    return out
